# baseline (speedup 1.0000x reference)
import jax
import jax.numpy as jnp
from jax.experimental import pallas as pl


_BLOCK = 8192


def _body(desc_ref, z_ref, w0_ref, b0_ref, w1_ref, out_ref):
    i = pl.program_id(0)
    q_bf = desc_ref[...].astype(jnp.bfloat16)
    z_row = z_ref[...].reshape(1, -1)
    tt = jax.lax.broadcasted_iota(jnp.int32, (8, 1), 0)
    m8 = (tt == z_row).astype(jnp.float32)
    mc = m8.T
    mc_bf = mc.astype(jnp.bfloat16)

    zpad = jnp.zeros((4, 128), jnp.float32)
    b0p = jnp.concatenate([b0_ref[...], zpad], axis=0).astype(jnp.bfloat16)
    w1f = w1_ref[...]
    w1hi4 = w1f.astype(jnp.bfloat16)
    w1lo4 = (w1f - w1hi4.astype(jnp.float32)).astype(jnp.bfloat16)
    zpad_bf = zpad.astype(jnp.bfloat16)
    w1hi = jnp.concatenate([w1hi4, zpad_bf], axis=0)
    w1lo = jnp.concatenate([w1lo4, zpad_bf], axis=0)

    acc = None
    for t in range(4):
        a_t = jnp.dot(q_bf, w0_ref[t].astype(jnp.bfloat16),
                      preferred_element_type=jnp.float32)
        term = a_t * mc[:, t:t + 1]
        acc = term if acc is None else acc + term
    acc = acc + jnp.dot(mc_bf, b0p, preferred_element_type=jnp.float32)
    th = jnp.tanh(acc)
    w1sel = (jnp.dot(mc_bf, w1hi, preferred_element_type=jnp.float32)
             + jnp.dot(mc_bf, w1lo, preferred_element_type=jnp.float32))
    evec = jnp.sum(th * w1sel, axis=0, keepdims=True)

    @pl.when(i == 0)
    def _():
        out_ref[...] = jnp.zeros_like(out_ref)

    out_ref[...] += evec


def kernel(descriptors, gradients, grad_index, positions, Z, box, W0, b0, W1, b1):
    n, d = descriptors.shape
    t, _, h = W0.shape
    block = min(_BLOCK, n)
    nb = n // block
    z3 = Z.astype(jnp.int32).reshape(nb, 1, block)

    out = pl.pallas_call(
        _body,
        grid=(nb,),
        in_specs=[
            pl.BlockSpec((block, d), lambda i: (i, 0)),
            pl.BlockSpec((1, 1, block), lambda i: (i, 0, 0)),
            pl.BlockSpec((t, d, h), lambda i: (0, 0, 0)),
            pl.BlockSpec((t, h), lambda i: (0, 0)),
            pl.BlockSpec((t, h), lambda i: (0, 0)),
        ],
        out_specs=pl.BlockSpec((1, h), lambda i: (0, 0)),
        out_shape=jax.ShapeDtypeStruct((1, h), jnp.float32),
    )(descriptors, z3, W0, b0, W1)
    return -(jnp.sum(out) + n * b1)


# speedup vs baseline: 1.0302x; 1.0302x over previous
import jax
import jax.numpy as jnp
from jax.experimental import pallas as pl
from jax.experimental.pallas import tpu as pltpu


_BLOCK = 4096


def _body(desc_ref, z_ref, w0_ref, b0_ref, w1_ref, out_ref):
    q_bf = desc_ref[...].astype(jnp.bfloat16)
    z_row = z_ref[...].reshape(1, -1)
    tt = jax.lax.broadcasted_iota(jnp.int32, (16, 1), 0)
    m16 = (tt % 8 == z_row).astype(jnp.float32)
    mc = m16.T
    mc_bf = mc.astype(jnp.bfloat16)

    zpad = jnp.zeros((4, 128), jnp.float32)
    b0p = jnp.concatenate([b0_ref[...], zpad], axis=0).astype(jnp.bfloat16)
    w1f = w1_ref[...]
    w1hi4 = w1f.astype(jnp.bfloat16)
    w1lo4 = (w1f - w1hi4.astype(jnp.float32)).astype(jnp.bfloat16)
    zpad_bf = zpad.astype(jnp.bfloat16)
    w1hilo = jnp.concatenate([w1hi4, zpad_bf, w1lo4, zpad_bf], axis=0)

    acc = jnp.dot(mc_bf[:, :8], b0p, preferred_element_type=jnp.float32)
    for t in range(4):
        qm_t = q_bf * mc_bf[:, t:t + 1]
        acc = acc + jnp.dot(qm_t, w0_ref[t].astype(jnp.bfloat16),
                            preferred_element_type=jnp.float32)
    th = jnp.tanh(acc)
    w1sel = jnp.dot(mc_bf, w1hilo, preferred_element_type=jnp.float32)
    out_ref[...] = jnp.sum(th * w1sel, axis=0, keepdims=True)[None]


def kernel(descriptors, gradients, grad_index, positions, Z, box, W0, b0, W1, b1):
    n, d = descriptors.shape
    t, _, h = W0.shape
    block = min(_BLOCK, n)
    nb = n // block
    z3 = Z.astype(jnp.int32).reshape(nb, 1, block)

    out = pl.pallas_call(
        _body,
        grid=(nb,),
        in_specs=[
            pl.BlockSpec((block, d), lambda i: (i, 0)),
            pl.BlockSpec((1, 1, block), lambda i: (i, 0, 0)),
            pl.BlockSpec((t, d, h), lambda i: (0, 0, 0)),
            pl.BlockSpec((t, h), lambda i: (0, 0)),
            pl.BlockSpec((t, h), lambda i: (0, 0)),
        ],
        out_specs=pl.BlockSpec((1, 1, h), lambda i: (i, 0, 0)),
        out_shape=jax.ShapeDtypeStruct((nb, 1, h), jnp.float32),
        compiler_params=pltpu.CompilerParams(
            dimension_semantics=("parallel",)),
    )(descriptors, z3, W0, b0, W1)
    return -(jnp.sum(out) + n * b1)


# speedup vs baseline: 1.0829x; 1.0511x over previous
import jax
import jax.numpy as jnp
from jax.experimental import pallas as pl
from jax.experimental.pallas import tpu as pltpu


_BLOCK = 4096
_STRIP = 512


def _body(desc_ref, z_ref, w0_ref, b0_ref, w1_ref, out_ref):
    z_row = z_ref[...].reshape(1, -1)
    tt = jax.lax.broadcasted_iota(jnp.int32, (16, 1), 0)
    m16 = (tt % 8 == z_row).astype(jnp.float32)
    mc = m16.T

    zpad = jnp.zeros((4, 128), jnp.float32)
    b0p = jnp.concatenate([b0_ref[...], zpad], axis=0).astype(jnp.bfloat16)
    w1f = w1_ref[...]
    w1hi4 = w1f.astype(jnp.bfloat16)
    w1lo4 = (w1f - w1hi4.astype(jnp.float32)).astype(jnp.bfloat16)
    zpad_bf = zpad.astype(jnp.bfloat16)
    w1hilo = jnp.concatenate([w1hi4, zpad_bf, w1lo4, zpad_bf], axis=0)
    w0_bf = [w0_ref[t].astype(jnp.bfloat16) for t in range(4)]

    block = desc_ref.shape[0]
    evec = jnp.zeros((1, 128), jnp.float32)
    for s in range(block // _STRIP):
        qs_bf = desc_ref[s * _STRIP:(s + 1) * _STRIP, :].astype(jnp.bfloat16)
        mcs = mc[s * _STRIP:(s + 1) * _STRIP, :]
        mcs_bf = mcs.astype(jnp.bfloat16)
        acc = jnp.dot(mcs_bf[:, :8], b0p, preferred_element_type=jnp.float32)
        for t in range(4):
            a_t = jnp.dot(qs_bf, w0_bf[t], preferred_element_type=jnp.float32)
            acc = acc + a_t * mcs[:, t:t + 1]
        th = jnp.tanh(acc)
        w1sel = jnp.dot(mcs_bf, w1hilo, preferred_element_type=jnp.float32)
        evec = evec + jnp.sum(th * w1sel, axis=0, keepdims=True)

    out_ref[...] = evec[None]


def kernel(descriptors, gradients, grad_index, positions, Z, box, W0, b0, W1, b1):
    n, d = descriptors.shape
    t, _, h = W0.shape
    block = min(_BLOCK, n)
    nb = n // block
    z3 = Z.astype(jnp.int32).reshape(nb, 1, block)

    out = pl.pallas_call(
        _body,
        grid=(nb,),
        in_specs=[
            pl.BlockSpec((block, d), lambda i: (i, 0)),
            pl.BlockSpec((1, 1, block), lambda i: (i, 0, 0)),
            pl.BlockSpec((t, d, h), lambda i: (0, 0, 0)),
            pl.BlockSpec((t, h), lambda i: (0, 0)),
            pl.BlockSpec((t, h), lambda i: (0, 0)),
        ],
        out_specs=pl.BlockSpec((1, 1, h), lambda i: (i, 0, 0)),
        out_shape=jax.ShapeDtypeStruct((nb, 1, h), jnp.float32),
        compiler_params=pltpu.CompilerParams(
            dimension_semantics=("parallel",)),
    )(descriptors, z3, W0, b0, W1)
    return -(jnp.sum(out) + n * b1)
